# Initial kernel scaffold; baseline (speedup 1.0000x reference)
import jax
import jax.numpy as jnp
from jax.experimental import pallas as pl


def kernel(x, edge_index, edge_values, batch, W1, b1, W2, b2, Wf, bf):
    raise NotImplementedError("write your pallas kernel here")


# speedup vs baseline: 5.2942x; 5.2942x over previous
import functools

import jax
import jax.numpy as jnp
from jax import lax
from jax.experimental import pallas as pl
from jax.experimental.pallas import tpu as pltpu
from jax.experimental.pallas import tpu_sc as plsc

N = 10000
NPAD = 10240
E = 160000
D_IN = 256
D_H = 512
NCH = 4
NC = 2
NS = 16
EPT = E // NS
B = 80
RPT = NPAD // NS
NB = NPAD // 1024


def _sc_degrees(dst, ev):
    mesh = plsc.VectorSubcoreMesh(core_axis_name="c", subcore_axis_name="s")

    @functools.partial(
        pl.kernel,
        out_type=(jax.ShapeDtypeStruct((NS, NPAD), jnp.float32),
                  jax.ShapeDtypeStruct((NS, NPAD), jnp.float32)),
        mesh=mesh,
        compiler_params=pltpu.CompilerParams(needs_layout_passes=False),
        scratch_types=[
            pltpu.VMEM((EPT,), jnp.int32),
            pltpu.VMEM((EPT,), jnp.float32),
            pltpu.VMEM((NPAD,), jnp.float32),
        ],
    )
    def deg_kernel(dst_hbm, ev_hbm, d1p_hbm, d2p_hbm, dbuf, evbuf, acc):
        cid = lax.axis_index("c")
        sid = lax.axis_index("s")

        def zbody(i, c):
            acc[pl.ds(i * 16, 16)] = jnp.zeros((16,), jnp.float32)
            return c
        lax.fori_loop(0, NPAD // 16, zbody, 0)

        base = sid * EPT
        pltpu.sync_copy(dst_hbm.at[pl.ds(base, EPT)], dbuf)
        lane0 = lax.iota(jnp.int32, 16) == 0

        @pl.when(cid == 0)
        def _():
            pltpu.sync_copy(ev_hbm.at[pl.ds(base, EPT)], evbuf)

            def body(i, c):
                idx = jnp.full((16,), i, jnp.int32)
                d16 = plsc.load_gather(dbuf, [idx])
                v16 = plsc.load_gather(evbuf, [idx])
                plsc.addupdate_scatter(acc, [d16], v16, mask=lane0)
                return c
            lax.fori_loop(0, EPT, body, 0)
            pltpu.sync_copy(acc, d1p_hbm.at[sid])

        @pl.when(cid == 1)
        def _():
            def body(i, c):
                idx = jnp.full((16,), i, jnp.int32)
                d16 = plsc.load_gather(dbuf, [idx])
                plsc.addupdate_scatter(acc, [d16], jnp.ones((16,), jnp.float32),
                                       mask=lane0)
                return c
            lax.fori_loop(0, EPT, body, 0)
            pltpu.sync_copy(acc, d2p_hbm.at[sid])

    return deg_kernel(dst, ev)


def _sc_aggregate(yflat, src, dst, ev, weighted):
    mesh = plsc.VectorSubcoreMesh(core_axis_name="c", subcore_axis_name="s")

    @functools.partial(
        pl.kernel,
        out_type=jax.ShapeDtypeStruct((NCH * NPAD, 128), jnp.float32),
        mesh=mesh,
        compiler_params=pltpu.CompilerParams(needs_layout_passes=False),
        scratch_types=[
            pltpu.VMEM((B,), jnp.int32),
            pltpu.VMEM((B,), jnp.int32),
            pltpu.VMEM((B,), jnp.int32),
            pltpu.VMEM((B,), jnp.float32),
            pltpu.VMEM((B, 128), jnp.float32),
            pltpu.VMEM_SHARED((NPAD, 128), jnp.float32),
            pltpu.SemaphoreType.DMA,
        ],
    )
    def agg_kernel(y_hbm, src_hbm, dst_hbm, ev_hbm, out_hbm,
                   sidx, sadj, didx, evb, rows, acc, sem):
        cid = lax.axis_index("c")
        sid = lax.axis_index("s")
        ebase = sid * EPT
        for k in range(NCH // NC):
            chunk = cid * (NCH // NC) + k
            cbase = chunk * NPAD
            pltpu.sync_copy(y_hbm.at[pl.ds(cbase + sid * RPT, RPT)],
                            acc.at[pl.ds(sid * RPT, RPT)])
            plsc.subcore_barrier()

            def body(j, c):
                b = ebase + j * B
                pltpu.sync_copy(src_hbm.at[pl.ds(b, B)], sidx)
                pltpu.sync_copy(dst_hbm.at[pl.ds(b, B)], didx)
                if weighted:
                    pltpu.sync_copy(ev_hbm.at[pl.ds(b, B)], evb)

                def adj(t, c2):
                    sadj[pl.ds(t * 16, 16)] = sidx[pl.ds(t * 16, 16)] + cbase
                    return c2
                lax.fori_loop(0, B // 16, adj, 0)

                pltpu.async_copy(y_hbm.at[sadj], rows, sem).wait()
                if weighted:
                    for i in range(B):
                        w = plsc.load_gather(evb, [jnp.full((16,), i, jnp.int32)])
                        for q in range(8):
                            sl = pl.ds(q * 16, 16)
                            rows[i, sl] = rows[i, sl] * w
                pltpu.sync_copy(rows, acc.at[didx], add=True)
                return c
            lax.fori_loop(0, EPT // B, body, 0)

            plsc.subcore_barrier()
            pltpu.sync_copy(acc.at[pl.ds(sid * RPT, RPT)],
                            out_hbm.at[pl.ds(cbase + sid * RPT, RPT)])
            plsc.subcore_barrier()

    return agg_kernel(yflat, src, dst, ev)


def _tc_stage1(xpad, W1, d1p, d2p):
    def body(x_ref, w1_ref, d1p_ref, d2p_ref, y1_ref, dinv_ref):
        xw = jnp.dot(x_ref[...], w1_ref[...], preferred_element_type=jnp.float32)
        dinv1 = lax.rsqrt(jnp.sum(d1p_ref[...], axis=0) + 1.0)
        dinv2 = lax.rsqrt(jnp.sum(d2p_ref[...], axis=0) + 1.0)
        y = xw * dinv1[:, None]
        for c in range(NCH):
            y1_ref[c] = y[:, c * 128:(c + 1) * 128]
        dinv_ref[0] = dinv1
        dinv_ref[1] = dinv2

    return pl.pallas_call(
        body,
        grid=(NB,),
        in_specs=[
            pl.BlockSpec((1024, D_IN), lambda i: (i, 0)),
            pl.BlockSpec((D_IN, D_H), lambda i: (0, 0)),
            pl.BlockSpec((NS, 1024), lambda i: (0, i)),
            pl.BlockSpec((NS, 1024), lambda i: (0, i)),
        ],
        out_specs=[
            pl.BlockSpec((NCH, 1024, 128), lambda i: (0, i, 0)),
            pl.BlockSpec((2, 1024), lambda i: (0, i)),
        ],
        out_shape=[
            jax.ShapeDtypeStruct((NCH, NPAD, 128), jnp.float32),
            jax.ShapeDtypeStruct((2, NPAD), jnp.float32),
        ],
    )(xpad, W1, d1p, d2p)


def _tc_stage2(g1, dinvs, b1, W2):
    def body(g1_ref, dinv_ref, b1_ref, w2_ref, y2_ref):
        s = jnp.concatenate([g1_ref[c] for c in range(NCH)], axis=1)
        h = jnp.maximum(s * dinv_ref[0][:, None] + b1_ref[...][None, :], 0.0)
        hw = jnp.dot(h, w2_ref[...], preferred_element_type=jnp.float32)
        y2 = hw * dinv_ref[1][:, None]
        for c in range(NCH):
            y2_ref[c] = y2[:, c * 128:(c + 1) * 128]

    return pl.pallas_call(
        body,
        grid=(NB,),
        in_specs=[
            pl.BlockSpec((NCH, 1024, 128), lambda i: (0, i, 0)),
            pl.BlockSpec((2, 1024), lambda i: (0, i)),
            pl.BlockSpec((D_H,), lambda i: (0,)),
            pl.BlockSpec((D_H, D_H), lambda i: (0, 0)),
        ],
        out_specs=pl.BlockSpec((NCH, 1024, 128), lambda i: (0, i, 0)),
        out_shape=jax.ShapeDtypeStruct((NCH, NPAD, 128), jnp.float32),
    )(g1, dinvs, b1, W2)


def _tc_stage3(g2, dinvs, b2, Wf, bf):
    def body(g2_ref, dinv_ref, b2_ref, wf_ref, bf_ref, out_ref, acc_ref):
        i = pl.program_id(0)
        s = jnp.concatenate([g2_ref[c] for c in range(NCH)], axis=1)
        h2 = jnp.maximum(s * dinv_ref[1][:, None] + b2_ref[...][None, :], 0.0)
        row = lax.broadcasted_iota(jnp.int32, (1024, 1), 0) + i * 1024
        h2 = jnp.where(row < N, h2, 0.0)
        psum = jnp.sum(h2, axis=0, keepdims=True)

        @pl.when(i == 0)
        def _():
            acc_ref[...] = psum

        @pl.when(i > 0)
        def _():
            acc_ref[...] += psum

        @pl.when(i == NB - 1)
        def _():
            pooled = acc_ref[...] * (1.0 / N)
            t = jnp.dot(pooled, wf_ref[...], preferred_element_type=jnp.float32)
            t = jnp.maximum(t + bf_ref[...][None, :], 0.0)
            m = jnp.max(t, axis=-1, keepdims=True)
            e = jnp.exp(t - m)
            out_ref[...] = e / jnp.sum(e, axis=-1, keepdims=True)

    return pl.pallas_call(
        body,
        grid=(NB,),
        in_specs=[
            pl.BlockSpec((NCH, 1024, 128), lambda i: (0, i, 0)),
            pl.BlockSpec((2, 1024), lambda i: (0, i)),
            pl.BlockSpec((D_H,), lambda i: (0,)),
            pl.BlockSpec((D_H, D_H), lambda i: (0, 0)),
            pl.BlockSpec((D_H,), lambda i: (0,)),
        ],
        out_specs=pl.BlockSpec((1, D_H), lambda i: (0, 0)),
        out_shape=jax.ShapeDtypeStruct((1, D_H), jnp.float32),
        scratch_shapes=[pltpu.VMEM((1, D_H), jnp.float32)],
    )(g2, dinvs, b2, Wf, bf)


def kernel(x, edge_index, edge_values, batch, W1, b1, W2, b2, Wf, bf):
    src = edge_index[0]
    dst = edge_index[1]
    xpad = jnp.pad(x, ((0, NPAD - N), (0, 0)))

    d1p, d2p = _sc_degrees(dst, edge_values)
    y1, dinvs = _tc_stage1(xpad, W1, d1p, d2p)
    g1 = _sc_aggregate(y1.reshape(NCH * NPAD, 128), src, dst, edge_values,
                       weighted=True)
    y2 = _tc_stage2(g1.reshape(NCH, NPAD, 128), dinvs, b1, W2)
    g2 = _sc_aggregate(y2.reshape(NCH * NPAD, 128), src, dst, edge_values,
                       weighted=False)
    return _tc_stage3(g2.reshape(NCH, NPAD, 128), dinvs, b2, Wf, bf)
